# baseline (speedup 1.0000x reference)
import functools

import jax
import jax.numpy as jnp
from jax import lax
from jax.experimental import pallas as pl
from jax.experimental.pallas import tpu as pltpu
from jax.experimental.pallas import tpu_sc as plsc

N = 100000
D = 256
G = 512
OUT = 2
NC = 2
NS = 16
NW = NC * NS
L = 16
NJ = D // L
SEG_PER_W = G // NW
CHUNK = 16
PA = 6256
BIG = 2 ** 30
NEG = -3.0e38

def _bf16r(v):
    u = plsc.bitcast(v, jnp.int32)
    r = (u + 0x7FFF + ((u >> 16) & 1)) & jnp.int32(-65536)
    return plsc.bitcast(r, jnp.float32)


_mesh = plsc.VectorSubcoreMesh(
    core_axis_name="c", subcore_axis_name="s", num_cores=NC, num_subcores=NS
)


@functools.partial(
    pl.kernel,
    out_type=jax.ShapeDtypeStruct((NW * OUT * L,), jnp.float32),
    mesh=_mesh,
    scratch_types=[
        pltpu.VMEM((PA + 8,), jnp.int32),
        pltpu.VMEM((G,), jnp.int32),
        pltpu.VMEM((G,), jnp.int32),
        pltpu.VMEM((NS * G,), jnp.int32),
        pltpu.VMEM((OUT * 2 * D,), jnp.float32),
        pltpu.VMEM((L,), jnp.float32),
        pltpu.VMEM((CHUNK * D,), jnp.float32),
        pltpu.VMEM((OUT * L,), jnp.float32),
        pltpu.VMEM_SHARED((2 * NS * G,), jnp.int32),
    ],
    compiler_params=pltpu.CompilerParams(needs_layout_passes=False),
)
def _sc_pool(x_hbm, batch_hbm, w_hbm, b_hbm, out_hbm,
             bbuf, starts, ends, tmp, wbuf, bvm, xbuf, obuf, sh):
    c = lax.axis_index("c")
    s = lax.axis_index("s")
    w = c * NS + s
    lanes = jnp.arange(L, dtype=jnp.int32)

    def init_body(i, _):
        starts[pl.ds(i * L, L)] = jnp.full((L,), BIG, jnp.int32)
        ends[pl.ds(i * L, L)] = jnp.zeros((L,), jnp.int32)
        return 0

    lax.fori_loop(0, G // L, init_body, 0)

    off = pl.multiple_of(jnp.minimum(s * PA, N - PA), 8)
    poff = pl.multiple_of(jnp.maximum(off - 8, 0), 8)
    pltpu.sync_copy(batch_hbm.at[pl.ds(off, PA)], bbuf.at[pl.ds(8, PA)])
    pltpu.sync_copy(batch_hbm.at[pl.ds(poff, 8)], bbuf.at[pl.ds(0, 8)])
    start_vreg = (s * PA - off) // L

    def pa_body(i, _):
        cur = bbuf[pl.ds(8 + i * L, L)]
        prev = bbuf[pl.ds(7 + i * L, L)]
        gpos = off + i * L + lanes
        isb = (cur != prev) | (gpos == 0)
        plsc.store_scatter(starts, [cur], gpos, mask=isb)
        plsc.store_scatter(ends, [prev], gpos, mask=isb & (gpos > 0))
        return 0

    lax.fori_loop(start_vreg, PA // L, pa_body, 0)

    @pl.when(s == NS - 1)
    def _():
        last_id = bbuf[pl.ds(8 + PA - L, L)][L - 1]
        plsc.store_scatter(
            ends,
            [jnp.full((L,), last_id, jnp.int32)],
            jnp.full((L,), N, jnp.int32),
            mask=lanes == 0,
        )

    pltpu.sync_copy(starts, sh.at[pl.ds(s * G, G)])
    pltpu.sync_copy(ends, sh.at[pl.ds((NS + s) * G, G)])
    plsc.subcore_barrier()

    mycols = w + NW * lanes

    pltpu.sync_copy(sh.at[pl.ds(0, NS * G)], tmp)

    def red_lo(t, lo):
        vals = plsc.load_gather(tmp, [t * G + mycols])
        return jnp.minimum(lo, vals)

    lo = lax.fori_loop(0, NS, red_lo, jnp.full((L,), BIG, jnp.int32))
    pltpu.sync_copy(sh.at[pl.ds(NS * G, NS * G)], tmp)

    def red_hi(t, hi):
        vals = plsc.load_gather(tmp, [t * G + mycols])
        return jnp.maximum(hi, vals)

    hi = lax.fori_loop(0, NS, red_hi, jnp.zeros((L,), jnp.int32))

    pltpu.sync_copy(w_hbm, wbuf)
    pltpu.sync_copy(b_hbm, bvm)
    bv = bvm[pl.ds(0, L)]
    b0 = bv[0]
    b1 = bv[1]

    def wround(j, _):
        wbuf[pl.ds(j * L, L)] = _bf16r(wbuf[pl.ds(j * L, L)])
        return 0

    lax.fori_loop(0, OUT * 2 * D // L, wround, 0)

    zf = jnp.zeros((L,), jnp.float32)
    nf = jnp.full((L,), NEG, jnp.float32)
    zi = jnp.zeros((L,), jnp.int32)

    def seg_body(k, carry):
        lg0, lg1 = carry
        sel = lanes == k
        lo_k = jnp.sum(jnp.where(sel, lo, zi))
        hi_k = jnp.sum(jnp.where(sel, hi, zi))

        def wcond(st):
            return st[0] < hi_k

        def wbody(st):
            cs = st[0]
            sa = jnp.minimum(cs, N - CHUNK)
            pltpu.sync_copy(
                x_hbm.at[pl.ds(pl.multiple_of(sa * D, D), CHUNK * D)], xbuf)
            sums = list(st[1:1 + NJ])
            maxs = list(st[1 + NJ:])
            for r in range(CHUNK):
                gp = sa + r
                validv = jnp.broadcast_to((gp >= cs) & (gp < hi_k), (L,))
                for j in range(NJ):
                    v = xbuf[pl.ds(r * D + j * L, L)]
                    sums[j] = sums[j] + jnp.where(validv, v, 0.0)
                    maxs[j] = jnp.maximum(maxs[j], jnp.where(validv, v, NEG))
            return (cs + CHUNK, *sums, *maxs)

        init = (lo_k, *([zf] * NJ), *([nf] * NJ))
        fin = lax.while_loop(wcond, wbody, init)

        cnt = jnp.maximum(hi_k - lo_k, 0)
        cntv = jnp.broadcast_to(
            jnp.maximum(cnt.astype(jnp.float32), 1.0), (L,))
        rin = jnp.full((L,), 1.0, jnp.float32) / cntv
        nonempty = jnp.broadcast_to(cnt > 0, (L,))
        acc0 = zf
        acc1 = zf
        for j in range(NJ):
            mean_j = _bf16r(fin[1 + j] * rin)
            max_j = _bf16r(jnp.where(nonempty, fin[1 + NJ + j], 0.0))
            acc0 = (acc0 + mean_j * wbuf[pl.ds(j * L, L)]
                    + max_j * wbuf[pl.ds(D + j * L, L)])
            acc1 = (acc1 + mean_j * wbuf[pl.ds(2 * D + j * L, L)]
                    + max_j * wbuf[pl.ds(3 * D + j * L, L)])
        l0 = jnp.sum(acc0) + b0
        l1 = jnp.sum(acc1) + b1
        lg0 = jnp.where(sel, jnp.broadcast_to(l0, (L,)), lg0)
        lg1 = jnp.where(sel, jnp.broadcast_to(l1, (L,)), lg1)
        return (lg0, lg1)

    lg0, lg1 = lax.fori_loop(0, SEG_PER_W, seg_body, (zf, zf))
    obuf[pl.ds(0, L)] = lg0
    obuf[pl.ds(L, L)] = lg1
    pltpu.sync_copy(obuf, out_hbm.at[pl.ds(w * OUT * L, OUT * L)])


def kernel(x, batch, W, b):
    x1 = x.reshape(-1)
    batch32 = batch.astype(jnp.int32)
    w1 = W.reshape(-1)
    b16 = jnp.zeros((L,), jnp.float32).at[:OUT].set(b)
    out = _sc_pool(x1, batch32, w1, b16)
    return out.reshape(NW, OUT, L).transpose(2, 0, 1).reshape(G, OUT)


# speedup vs baseline: 1.4537x; 1.4537x over previous
import functools

import jax
import jax.numpy as jnp
from jax import lax
from jax.experimental import pallas as pl
from jax.experimental.pallas import tpu as pltpu
from jax.experimental.pallas import tpu_sc as plsc

N = 100000
D = 256
G = 512
OUT = 2
NC = 2
NS = 16
NW = NC * NS
L = 16
NJ = D // L
SEG_PER_W = G // NW
CHUNK = 128
SUB = 32
PA = 6256
BIG = 2 ** 30
NEG = -3.0e38

def _bf16r(v):
    u = plsc.bitcast(v, jnp.int32)
    r = (u + 0x7FFF + ((u >> 16) & 1)) & jnp.int32(-65536)
    return plsc.bitcast(r, jnp.float32)


_mesh = plsc.VectorSubcoreMesh(
    core_axis_name="c", subcore_axis_name="s", num_cores=NC, num_subcores=NS
)


@functools.partial(
    pl.kernel,
    out_type=jax.ShapeDtypeStruct((NW * OUT * L,), jnp.float32),
    mesh=_mesh,
    scratch_types=[
        pltpu.VMEM((PA + 8,), jnp.int32),
        pltpu.VMEM((G,), jnp.int32),
        pltpu.VMEM((G,), jnp.int32),
        pltpu.VMEM((NS * G,), jnp.int32),
        pltpu.VMEM((OUT * 2 * D,), jnp.float32),
        pltpu.VMEM((L,), jnp.float32),
        pltpu.VMEM((CHUNK * D,), jnp.float32),
        pltpu.VMEM((OUT * L,), jnp.float32),
        pltpu.VMEM_SHARED((2 * NS * G,), jnp.int32),
    ],
    compiler_params=pltpu.CompilerParams(needs_layout_passes=False),
)
def _sc_pool(x_hbm, batch_hbm, w_hbm, b_hbm, out_hbm,
             bbuf, starts, ends, tmp, wbuf, bvm, xbuf, obuf, sh):
    c = lax.axis_index("c")
    s = lax.axis_index("s")
    w = c * NS + s
    lanes = jnp.arange(L, dtype=jnp.int32)

    def init_body(i, _):
        starts[pl.ds(i * L, L)] = jnp.full((L,), BIG, jnp.int32)
        ends[pl.ds(i * L, L)] = jnp.zeros((L,), jnp.int32)
        return 0

    lax.fori_loop(0, G // L, init_body, 0)

    off = pl.multiple_of(jnp.minimum(s * PA, N - PA), 8)
    poff = pl.multiple_of(jnp.maximum(off - 8, 0), 8)
    pltpu.sync_copy(batch_hbm.at[pl.ds(off, PA)], bbuf.at[pl.ds(8, PA)])
    pltpu.sync_copy(batch_hbm.at[pl.ds(poff, 8)], bbuf.at[pl.ds(0, 8)])
    start_vreg = (s * PA - off) // L

    def pa_body(i, _):
        cur = bbuf[pl.ds(8 + i * L, L)]
        prev = bbuf[pl.ds(7 + i * L, L)]
        gpos = off + i * L + lanes
        isb = (cur != prev) | (gpos == 0)
        plsc.store_scatter(starts, [cur], gpos, mask=isb)
        plsc.store_scatter(ends, [prev], gpos, mask=isb & (gpos > 0))
        return 0

    lax.fori_loop(start_vreg, PA // L, pa_body, 0)

    @pl.when(s == NS - 1)
    def _():
        last_id = bbuf[pl.ds(8 + PA - L, L)][L - 1]
        plsc.store_scatter(
            ends,
            [jnp.full((L,), last_id, jnp.int32)],
            jnp.full((L,), N, jnp.int32),
            mask=lanes == 0,
        )

    pltpu.sync_copy(starts, sh.at[pl.ds(s * G, G)])
    pltpu.sync_copy(ends, sh.at[pl.ds((NS + s) * G, G)])
    plsc.subcore_barrier()

    mycols = w + NW * lanes

    pltpu.sync_copy(sh.at[pl.ds(0, NS * G)], tmp)

    def red_lo(t, lo):
        vals = plsc.load_gather(tmp, [t * G + mycols])
        return jnp.minimum(lo, vals)

    lo = lax.fori_loop(0, NS, red_lo, jnp.full((L,), BIG, jnp.int32))
    pltpu.sync_copy(sh.at[pl.ds(NS * G, NS * G)], tmp)

    def red_hi(t, hi):
        vals = plsc.load_gather(tmp, [t * G + mycols])
        return jnp.maximum(hi, vals)

    hi = lax.fori_loop(0, NS, red_hi, jnp.zeros((L,), jnp.int32))

    pltpu.sync_copy(w_hbm, wbuf)
    pltpu.sync_copy(b_hbm, bvm)
    bv = bvm[pl.ds(0, L)]
    b0 = bv[0]
    b1 = bv[1]

    def wround(j, _):
        wbuf[pl.ds(j * L, L)] = _bf16r(wbuf[pl.ds(j * L, L)])
        return 0

    lax.fori_loop(0, OUT * 2 * D // L, wround, 0)

    zf = jnp.zeros((L,), jnp.float32)
    nf = jnp.full((L,), NEG, jnp.float32)
    zi = jnp.zeros((L,), jnp.int32)

    def seg_body(k, carry):
        lg0, lg1 = carry
        sel = lanes == k
        lo_k = jnp.sum(jnp.where(sel, lo, zi))
        hi_k = jnp.sum(jnp.where(sel, hi, zi))

        def wcond(st):
            return st[0] < hi_k

        def wbody(st):
            cs = st[0]
            sa = jnp.minimum(cs, N - CHUNK)
            pltpu.sync_copy(
                x_hbm.at[pl.ds(pl.multiple_of(sa * D, D), CHUNK * D)], xbuf)
            nsub = jnp.minimum((hi_k - sa + SUB - 1) // SUB, CHUNK // SUB)

            def sub_body(q, sst):
                ssums = list(sst[:NJ])
                smaxs = list(sst[NJ:])
                for r in range(SUB):
                    gp = sa + q * SUB + r
                    validv = jnp.broadcast_to(
                        (gp >= cs) & (gp < hi_k), (L,))
                    for j in range(NJ):
                        v = xbuf[pl.ds((q * SUB + r) * D + j * L, L)]
                        ssums[j] = ssums[j] + jnp.where(validv, v, 0.0)
                        smaxs[j] = jnp.maximum(
                            smaxs[j], jnp.where(validv, v, NEG))
                return (*ssums, *smaxs)

            fin_sub = lax.fori_loop(
                0, nsub, sub_body, (*st[1:1 + NJ], *st[1 + NJ:]))
            return (cs + CHUNK, *fin_sub)

        init = (lo_k, *([zf] * NJ), *([nf] * NJ))
        fin = lax.while_loop(wcond, wbody, init)

        cnt = jnp.maximum(hi_k - lo_k, 0)
        cntv = jnp.broadcast_to(
            jnp.maximum(cnt.astype(jnp.float32), 1.0), (L,))
        rin = jnp.full((L,), 1.0, jnp.float32) / cntv
        nonempty = jnp.broadcast_to(cnt > 0, (L,))
        acc0 = zf
        acc1 = zf
        for j in range(NJ):
            mean_j = _bf16r(fin[1 + j] * rin)
            max_j = _bf16r(jnp.where(nonempty, fin[1 + NJ + j], 0.0))
            acc0 = (acc0 + mean_j * wbuf[pl.ds(j * L, L)]
                    + max_j * wbuf[pl.ds(D + j * L, L)])
            acc1 = (acc1 + mean_j * wbuf[pl.ds(2 * D + j * L, L)]
                    + max_j * wbuf[pl.ds(3 * D + j * L, L)])
        l0 = jnp.sum(acc0) + b0
        l1 = jnp.sum(acc1) + b1
        lg0 = jnp.where(sel, jnp.broadcast_to(l0, (L,)), lg0)
        lg1 = jnp.where(sel, jnp.broadcast_to(l1, (L,)), lg1)
        return (lg0, lg1)

    lg0, lg1 = lax.fori_loop(0, SEG_PER_W, seg_body, (zf, zf))
    obuf[pl.ds(0, L)] = lg0
    obuf[pl.ds(L, L)] = lg1
    pltpu.sync_copy(obuf, out_hbm.at[pl.ds(w * OUT * L, OUT * L)])


def kernel(x, batch, W, b):
    x1 = x.reshape(-1)
    batch32 = batch.astype(jnp.int32)
    w1 = W.reshape(-1)
    b16 = jnp.zeros((L,), jnp.float32).at[:OUT].set(b)
    out = _sc_pool(x1, batch32, w1, b16)
    return out.reshape(NW, OUT, L).transpose(2, 0, 1).reshape(G, OUT)


# speedup vs baseline: 1.5262x; 1.0499x over previous
import functools

import jax
import jax.numpy as jnp
from jax import lax
from jax.experimental import pallas as pl
from jax.experimental.pallas import tpu as pltpu
from jax.experimental.pallas import tpu_sc as plsc

N = 100000
D = 256
G = 512
OUT = 2
NC = 2
NS = 16
NW = NC * NS
L = 16
NJ = D // L
SEG_PER_W = G // NW
CHUNK = 256
SUB = 32
PA = 6256
BIG = 2 ** 30
NEG = -3.0e38

def _bf16r(v):
    u = plsc.bitcast(v, jnp.int32)
    r = (u + 0x7FFF + ((u >> 16) & 1)) & jnp.int32(-65536)
    return plsc.bitcast(r, jnp.float32)


_mesh = plsc.VectorSubcoreMesh(
    core_axis_name="c", subcore_axis_name="s", num_cores=NC, num_subcores=NS
)


@functools.partial(
    pl.kernel,
    out_type=jax.ShapeDtypeStruct((NW * OUT * L,), jnp.float32),
    mesh=_mesh,
    scratch_types=[
        pltpu.VMEM((PA + 8,), jnp.int32),
        pltpu.VMEM((G,), jnp.int32),
        pltpu.VMEM((G,), jnp.int32),
        pltpu.VMEM((NS * G,), jnp.int32),
        pltpu.VMEM((OUT * 2 * D,), jnp.float32),
        pltpu.VMEM((L,), jnp.float32),
        pltpu.VMEM((CHUNK * D,), jnp.float32),
        pltpu.VMEM((OUT * L,), jnp.float32),
        pltpu.VMEM_SHARED((2 * NS * G,), jnp.int32),
    ],
    compiler_params=pltpu.CompilerParams(needs_layout_passes=False),
)
def _sc_pool(x_hbm, batch_hbm, w_hbm, b_hbm, out_hbm,
             bbuf, starts, ends, tmp, wbuf, bvm, xbuf, obuf, sh):
    c = lax.axis_index("c")
    s = lax.axis_index("s")
    w = c * NS + s
    lanes = jnp.arange(L, dtype=jnp.int32)

    def init_body(i, _):
        starts[pl.ds(i * L, L)] = jnp.full((L,), BIG, jnp.int32)
        ends[pl.ds(i * L, L)] = jnp.zeros((L,), jnp.int32)
        return 0

    lax.fori_loop(0, G // L, init_body, 0)

    off = pl.multiple_of(jnp.minimum(s * PA, N - PA), 8)
    poff = pl.multiple_of(jnp.maximum(off - 8, 0), 8)
    pltpu.sync_copy(batch_hbm.at[pl.ds(off, PA)], bbuf.at[pl.ds(8, PA)])
    pltpu.sync_copy(batch_hbm.at[pl.ds(poff, 8)], bbuf.at[pl.ds(0, 8)])
    start_vreg = (s * PA - off) // L

    def pa_body(i, _):
        cur = bbuf[pl.ds(8 + i * L, L)]
        prev = bbuf[pl.ds(7 + i * L, L)]
        gpos = off + i * L + lanes
        isb = (cur != prev) | (gpos == 0)
        plsc.store_scatter(starts, [cur], gpos, mask=isb)
        plsc.store_scatter(ends, [prev], gpos, mask=isb & (gpos > 0))
        return 0

    lax.fori_loop(start_vreg, PA // L, pa_body, 0)

    @pl.when(s == NS - 1)
    def _():
        last_id = bbuf[pl.ds(8 + PA - L, L)][L - 1]
        plsc.store_scatter(
            ends,
            [jnp.full((L,), last_id, jnp.int32)],
            jnp.full((L,), N, jnp.int32),
            mask=lanes == 0,
        )

    pltpu.sync_copy(starts, sh.at[pl.ds(s * G, G)])
    pltpu.sync_copy(ends, sh.at[pl.ds((NS + s) * G, G)])
    plsc.subcore_barrier()

    mycols = w + NW * lanes

    pltpu.sync_copy(sh.at[pl.ds(0, NS * G)], tmp)

    def red_lo(t, lo):
        vals = plsc.load_gather(tmp, [t * G + mycols])
        return jnp.minimum(lo, vals)

    lo = lax.fori_loop(0, NS, red_lo, jnp.full((L,), BIG, jnp.int32))
    pltpu.sync_copy(sh.at[pl.ds(NS * G, NS * G)], tmp)

    def red_hi(t, hi):
        vals = plsc.load_gather(tmp, [t * G + mycols])
        return jnp.maximum(hi, vals)

    hi = lax.fori_loop(0, NS, red_hi, jnp.zeros((L,), jnp.int32))

    pltpu.sync_copy(w_hbm, wbuf)
    pltpu.sync_copy(b_hbm, bvm)
    bv = bvm[pl.ds(0, L)]
    b0 = bv[0]
    b1 = bv[1]

    def wround(j, _):
        wbuf[pl.ds(j * L, L)] = _bf16r(wbuf[pl.ds(j * L, L)])
        return 0

    lax.fori_loop(0, OUT * 2 * D // L, wround, 0)

    zf = jnp.zeros((L,), jnp.float32)
    nf = jnp.full((L,), NEG, jnp.float32)
    zi = jnp.zeros((L,), jnp.int32)

    def seg_body(k, carry):
        lg0, lg1 = carry
        sel = lanes == k
        lo_k = jnp.sum(jnp.where(sel, lo, zi))
        hi_k = jnp.sum(jnp.where(sel, hi, zi))

        def wcond(st):
            return st[0] < hi_k

        def wbody(st):
            cs = st[0]
            sa = jnp.minimum(cs, N - CHUNK)
            pltpu.sync_copy(
                x_hbm.at[pl.ds(pl.multiple_of(sa * D, D), CHUNK * D)], xbuf)
            nsub = jnp.minimum((hi_k - sa + SUB - 1) // SUB, CHUNK // SUB)

            def sub_body(q, sst):
                ssums = list(sst[:NJ])
                smaxs = list(sst[NJ:])
                for r in range(SUB):
                    gp = sa + q * SUB + r
                    validv = jnp.broadcast_to(
                        (gp >= cs) & (gp < hi_k), (L,))
                    for j in range(NJ):
                        v = xbuf[pl.ds((q * SUB + r) * D + j * L, L)]
                        ssums[j] = ssums[j] + jnp.where(validv, v, 0.0)
                        smaxs[j] = jnp.maximum(
                            smaxs[j], jnp.where(validv, v, NEG))
                return (*ssums, *smaxs)

            fin_sub = lax.fori_loop(
                0, nsub, sub_body, (*st[1:1 + NJ], *st[1 + NJ:]))
            return (cs + CHUNK, *fin_sub)

        init = (lo_k, *([zf] * NJ), *([nf] * NJ))
        fin = lax.while_loop(wcond, wbody, init)

        cnt = jnp.maximum(hi_k - lo_k, 0)
        cntv = jnp.broadcast_to(
            jnp.maximum(cnt.astype(jnp.float32), 1.0), (L,))
        rin = jnp.full((L,), 1.0, jnp.float32) / cntv
        nonempty = jnp.broadcast_to(cnt > 0, (L,))
        acc0 = zf
        acc1 = zf
        for j in range(NJ):
            mean_j = _bf16r(fin[1 + j] * rin)
            max_j = _bf16r(jnp.where(nonempty, fin[1 + NJ + j], 0.0))
            acc0 = (acc0 + mean_j * wbuf[pl.ds(j * L, L)]
                    + max_j * wbuf[pl.ds(D + j * L, L)])
            acc1 = (acc1 + mean_j * wbuf[pl.ds(2 * D + j * L, L)]
                    + max_j * wbuf[pl.ds(3 * D + j * L, L)])
        l0 = jnp.sum(acc0) + b0
        l1 = jnp.sum(acc1) + b1
        lg0 = jnp.where(sel, jnp.broadcast_to(l0, (L,)), lg0)
        lg1 = jnp.where(sel, jnp.broadcast_to(l1, (L,)), lg1)
        return (lg0, lg1)

    lg0, lg1 = lax.fori_loop(0, SEG_PER_W, seg_body, (zf, zf))
    obuf[pl.ds(0, L)] = lg0
    obuf[pl.ds(L, L)] = lg1
    pltpu.sync_copy(obuf, out_hbm.at[pl.ds(w * OUT * L, OUT * L)])


def kernel(x, batch, W, b):
    x1 = x.reshape(-1)
    batch32 = batch.astype(jnp.int32)
    w1 = W.reshape(-1)
    b16 = jnp.zeros((L,), jnp.float32).at[:OUT].set(b)
    out = _sc_pool(x1, batch32, w1, b16)
    return out.reshape(NW, OUT, L).transpose(2, 0, 1).reshape(G, OUT)
